# baseline (speedup 1.0000x reference)
import functools

import jax
import jax.numpy as jnp
from jax.experimental import pallas as pl


def _body(x_ref, m_ref, o_ref, *, hw):
    x = x_ref[...]
    m = m_ref[0]
    is_fg = m >= 0.5
    fg = is_fg.astype(jnp.float32)

    n_fg = jnp.sum(fg)
    n_bg = hw - n_fg

    xf = x * fg
    s_all = jnp.sum(x, axis=1, keepdims=True)
    s_fg = jnp.sum(xf, axis=1, keepdims=True)
    q_all = jnp.sum(x * x, axis=1, keepdims=True)
    q_fg = jnp.sum(x * xf, axis=1, keepdims=True)

    mu_fg = s_fg / n_fg
    mu_bg = (s_all - s_fg) / n_bg
    var_fg = (q_fg - n_fg * mu_fg * mu_fg) / (n_fg - 1.0)
    var_bg = ((q_all - q_fg) - n_bg * mu_bg * mu_bg) / (n_bg - 1.0)
    scale = jnp.sqrt(var_fg) / (jnp.sqrt(var_bg) + 1e-8)

    new_bg = (x - mu_bg) * scale + mu_fg
    o_ref[...] = jnp.where(is_fg, x, new_bg)


def kernel(x, mask):
    b, c, h, w = x.shape
    hw = h * w
    x2 = x.reshape(b * c, hw)
    m2 = mask.reshape(b, 1, hw)

    r = 8 if c % 8 == 0 else 1
    grid = (b * c) // r
    rows_per_b = c // r

    out = pl.pallas_call(
        functools.partial(_body, hw=float(hw)),
        grid=(grid,),
        in_specs=[
            pl.BlockSpec((r, hw), lambda i: (i, 0)),
            pl.BlockSpec((1, 1, hw), lambda i: (i // rows_per_b, 0, 0)),
        ],
        out_specs=pl.BlockSpec((r, hw), lambda i: (i, 0)),
        out_shape=jax.ShapeDtypeStruct((b * c, hw), x.dtype),
    )(x2, m2)
    return out.reshape(b, c, h, w)
